# Initial kernel scaffold; baseline (speedup 1.0000x reference)
import jax
import jax.numpy as jnp
from jax.experimental import pallas as pl


def kernel(nodes, relations, triples):
    raise NotImplementedError("write your pallas kernel here")


# speedup vs baseline: 25.1677x; 25.1677x over previous
import functools

import jax
import jax.numpy as jnp
from jax import lax
from jax.experimental import pallas as pl
from jax.experimental.pallas import tpu as pltpu
from jax.experimental.pallas import tpu_sc as plsc

NNODES = 100000
NREL = 200
H = 16
E = 3200000

NC = 2
NS = 16
NW = NC * NS
SUB = 128
CHUNK = 1024
NSUB = CHUNK // SUB
NCHUNKS = E // CHUNK
BASE_CH, EXTRA = divmod(NCHUNKS, NW)
NBLK = CHUNK // 16


def _body(nodes_hbm, rel_hbm, s_hbm, p_hbm, o_hbm, out_hbm,
          rel_v, sidx, oidx, pidx, s_rows, o_rows, out_v, sem):
    cid = lax.axis_index("c")
    sid = lax.axis_index("s")
    wid = sid * NC + cid
    nchunks = BASE_CH + jnp.where(wid < EXTRA, 1, 0)

    pltpu.sync_copy(rel_hbm, rel_v)

    def chunk_body(t, carry):
        c = wid + NW * t
        base = c * CHUNK
        pltpu.sync_copy(s_hbm.at[pl.ds(c * NSUB, NSUB)], sidx)
        pltpu.sync_copy(o_hbm.at[pl.ds(c * NSUB, NSUB)], oidx)
        pltpu.sync_copy(p_hbm.at[pl.ds(base, CHUNK)], pidx)
        handles = []
        for j in range(NSUB):
            handles.append(pltpu.async_copy(
                nodes_hbm.at[sidx.at[j]],
                s_rows.at[pl.ds(j * SUB, SUB)], sem))
            handles.append(pltpu.async_copy(
                nodes_hbm.at[oidx.at[j]],
                o_rows.at[pl.ds(j * SUB, SUB)], sem))
        for h in handles:
            h.wait()

        def blk(tb, carry2):
            rbase = tb * 16
            rows = rbase + lax.iota(jnp.int32, 16)
            p_ids = pidx[pl.ds(rbase, 16)]
            acc = jnp.zeros(16, jnp.float32)
            for h in range(H):
                hcol = jnp.full((16,), h, jnp.int32)
                sv = plsc.load_gather(s_rows, [rows, hcol])
                ov = plsc.load_gather(o_rows, [rows, hcol])
                pv = plsc.load_gather(rel_v, [p_ids, hcol])
                acc = acc + sv * pv * ov
            out_v[pl.ds(rbase, 16)] = acc
            return carry2

        lax.fori_loop(0, NBLK, blk, 0)
        pltpu.sync_copy(out_v, out_hbm.at[pl.ds(base, CHUNK)])
        return carry

    lax.fori_loop(0, nchunks, chunk_body, 0)


@jax.jit
def kernel(nodes, relations, triples):
    s = triples[:, 0].reshape(E // SUB, SUB)
    p = triples[:, 1]
    o = triples[:, 2].reshape(E // SUB, SUB)

    mesh = plsc.VectorSubcoreMesh(core_axis_name="c", subcore_axis_name="s")
    run = pl.kernel(
        _body,
        out_type=jax.ShapeDtypeStruct((E,), jnp.float32),
        mesh=mesh,
        compiler_params=pltpu.CompilerParams(needs_layout_passes=False,
                                              use_tc_tiling_on_sc=False),
        scratch_types=[
            pltpu.VMEM((NREL, H), jnp.float32),
            pltpu.VMEM((NSUB, SUB), jnp.int32),
            pltpu.VMEM((NSUB, SUB), jnp.int32),
            pltpu.VMEM((CHUNK,), jnp.int32),
            pltpu.VMEM((CHUNK, H), jnp.float32),
            pltpu.VMEM((CHUNK, H), jnp.float32),
            pltpu.VMEM((CHUNK,), jnp.float32),
            pltpu.SemaphoreType.DMA,
        ],
    )
    return run(nodes, relations, s, p, o)


# speedup vs baseline: 34.7175x; 1.3794x over previous
import jax
import jax.numpy as jnp
from jax import lax
from jax.experimental import pallas as pl
from jax.experimental.pallas import tpu as pltpu
from jax.experimental.pallas import tpu_sc as plsc

NNODES = 100000
NREL = 200
H = 16
E = 3200000

NC = 2
NS = 16
NW = NC * NS
SUB = 128
CHUNK = 1024
NSUB = CHUNK // SUB
NCHUNKS = E // CHUNK
BASE_CH, EXTRA = divmod(NCHUNKS, NW)
NBLK = CHUNK // 16
TMAX = BASE_CH + 1
UNROLL = 6
NT2 = -(-TMAX // UNROLL)


def _body(nodes_hbm, rel_hbm, idx_hbm, out_hbm,
          rel_v, idx_v, s_rows, o_rows, out_v,
          sem_i0, sem_i1, sem_i2, sem_g0, sem_g1, sem_o0, sem_o1):
    cid = lax.axis_index("c")
    sid = lax.axis_index("s")
    wid = sid * NC + cid
    nchunks = BASE_CH + jnp.where(wid < EXTRA, 1, 0)
    sem_i = (sem_i0, sem_i1, sem_i2)
    sem_g = (sem_g0, sem_g1)
    sem_o = (sem_o0, sem_o1)

    pltpu.sync_copy(rel_hbm, rel_v)

    def fire_idx(t, p3):
        @pl.when(t < nchunks)
        def _():
            c = wid + NW * t
            pltpu.async_copy(idx_hbm.at[c], idx_v.at[p3], sem_i[p3])

    def wait_idx(t, p3):
        @pl.when(t < nchunks)
        def _():
            pltpu.make_async_copy(idx_hbm.at[0], idx_v.at[p3], sem_i[p3]).wait()

    def fire_gathers(t, p3, p2):
        @pl.when(t < nchunks)
        def _():
            for j in range(NSUB):
                pltpu.async_copy(nodes_hbm.at[idx_v.at[p3, j]],
                                 s_rows.at[p2, pl.ds(j * SUB, SUB)], sem_g[p2])
                pltpu.async_copy(nodes_hbm.at[idx_v.at[p3, NSUB + j]],
                                 o_rows.at[p2, pl.ds(j * SUB, SUB)], sem_g[p2])

    def drain_gathers(t, p2):
        @pl.when(t < nchunks)
        def _():
            dummy = nodes_hbm.at[pl.ds(0, CHUNK)]
            pltpu.make_async_copy(dummy, s_rows.at[p2], sem_g[p2]).wait()
            pltpu.make_async_copy(dummy, o_rows.at[p2], sem_g[p2]).wait()

    def drain_out(t, p2):
        @pl.when(jnp.logical_and(t >= 0, t < nchunks))
        def _():
            pltpu.make_async_copy(out_hbm.at[pl.ds(0, CHUNK)],
                                  out_v.at[p2], sem_o[p2]).wait()

    def compute(t, p3, p2):
        @pl.when(t < nchunks)
        def _():
            c = wid + NW * t

            def blk(tb, carry):
                rbase = tb * 16
                rows = rbase + lax.iota(jnp.int32, 16)
                p_ids = idx_v[p3, 2 * NSUB + tb // 8, pl.ds((tb % 8) * 16, 16)]
                acc = jnp.zeros(16, jnp.float32)
                for h in range(H):
                    hcol = jnp.full((16,), h, jnp.int32)
                    sv = plsc.load_gather(s_rows.at[p2], [rows, hcol])
                    ov = plsc.load_gather(o_rows.at[p2], [rows, hcol])
                    pv = plsc.load_gather(rel_v, [p_ids, hcol])
                    acc = acc + sv * pv * ov
                out_v[p2, pl.ds(rbase, 16)] = acc
                return carry

            lax.fori_loop(0, NBLK, blk, 0)
            pltpu.async_copy(out_v.at[p2],
                             out_hbm.at[pl.ds(c * CHUNK, CHUNK)], sem_o[p2])

    fire_idx(0, 0)
    fire_idx(1, 1)
    wait_idx(0, 0)
    fire_gathers(0, 0, 0)

    def t2_body(t2, carry):
        tb0 = t2 * UNROLL
        for u in range(UNROLL):
            t = tb0 + u
            p3, p2 = u % 3, u % 2
            fire_idx(t + 2, (u + 2) % 3)
            wait_idx(t + 1, (u + 1) % 3)
            fire_gathers(t + 1, (u + 1) % 3, (u + 1) % 2)
            drain_gathers(t, p2)
            drain_out(t - 2, p2)
            compute(t, p3, p2)
        return carry

    lax.fori_loop(0, NT2, t2_body, 0)


@jax.jit
def kernel(nodes, relations, triples):
    s = triples[:, 0].reshape(NCHUNKS, NSUB, SUB)
    p = triples[:, 1].reshape(NCHUNKS, NSUB, SUB)
    o = triples[:, 2].reshape(NCHUNKS, NSUB, SUB)
    idx = jnp.concatenate([s, o, p], axis=1)

    mesh = plsc.VectorSubcoreMesh(core_axis_name="c", subcore_axis_name="s")
    run = pl.kernel(
        _body,
        out_type=jax.ShapeDtypeStruct((E,), jnp.float32),
        mesh=mesh,
        compiler_params=pltpu.CompilerParams(needs_layout_passes=False,
                                             use_tc_tiling_on_sc=False),
        scratch_types=[
            pltpu.VMEM((NREL, H), jnp.float32),
            pltpu.VMEM((3, 3 * NSUB, SUB), jnp.int32),
            pltpu.VMEM((2, CHUNK, H), jnp.float32),
            pltpu.VMEM((2, CHUNK, H), jnp.float32),
            pltpu.VMEM((2, CHUNK), jnp.float32),
            pltpu.SemaphoreType.DMA,
            pltpu.SemaphoreType.DMA,
            pltpu.SemaphoreType.DMA,
            pltpu.SemaphoreType.DMA,
            pltpu.SemaphoreType.DMA,
            pltpu.SemaphoreType.DMA,
            pltpu.SemaphoreType.DMA,
        ],
    )
    return run(nodes, relations, idx)
